# baseline (speedup 1.0000x reference)
import jax
import jax.numpy as jnp
from jax import lax
from jax.experimental import pallas as pl
from jax.experimental.pallas import tpu as pltpu
from jax.experimental.pallas import tpu_sc as plsc

N = 10000
E = 320000
D = 128
H = 64
NC = 2
NS = 16
C = 128
NB = 2
CPG = C * NB
EPT = -(-E // (NS * CPG)) * CPG
EPAD = EPT * NS
STEPS = EPT // C
GROUPS = STEPS // NB
NACC = ((N + 1 + 127) // 128) * 128
ZR = NACC // NS


def _sc_body(xsplit, src2d, dst2d, zacc, zdeg, ones_h,
             acc_out, deg_out,
             shared_acc, shared_deg,
             src_all, dst_all, rows, ones_v, zdeg_v, zacc_v, gsem, ssem, dsem):
    cid = lax.axis_index("c")
    sid = lax.axis_index("s")

    pltpu.sync_copy(zacc, zacc_v)
    pltpu.sync_copy(zdeg, zdeg_v)
    zbase = sid * ZR
    for i, sz in ((0, 128), (128, 128), (256, 128), (384, 128), (512, 120)):
        pltpu.sync_copy(zacc_v.at[pl.ds(0, sz)],
                        shared_acc.at[pl.ds(zbase + i, sz)])
        pltpu.sync_copy(zdeg_v.at[pl.ds(0, sz)],
                        shared_deg.at[pl.ds(zbase + i, sz)])
    pltpu.sync_copy(ones_h, ones_v)

    pltpu.sync_copy(src2d.at[pl.ds((cid * NS + sid) * STEPS, STEPS)], src_all)
    pltpu.sync_copy(dst2d.at[pl.ds(sid * STEPS, STEPS)], dst_all)

    for b in range(NB):
        pltpu.async_copy(xsplit.at[src_all.at[b]], rows.at[b], gsem.at[b])

    plsc.subcore_barrier()

    def group(k, carry):
        for b in range(NB):
            j = k * NB + b
            pb = (b + NB - 1) % NB
            pltpu.make_async_copy(xsplit.at[src_all.at[j]], rows.at[b],
                                  gsem.at[b]).wait()
            pltpu.async_copy(ones_v, shared_deg.at[dst_all.at[j]],
                             dsem.at[b], add=True)

            @pl.when(j > 0)
            def _():
                jp = j - 1
                pltpu.make_async_copy(
                    ones_v, shared_deg.at[dst_all.at[jp]],
                    dsem.at[pb]).wait()

                @pl.when(jp + NB < STEPS)
                def _():
                    pltpu.async_copy(xsplit.at[src_all.at[jp + NB]],
                                     rows.at[pb], gsem.at[pb])
        return carry

    lax.fori_loop(0, GROUPS, group, 0)
    last = NB - 1
    pltpu.make_async_copy(ones_v, shared_deg.at[dst_all.at[STEPS - 1]],
                          dsem.at[last]).wait()
    plsc.subcore_barrier()

    pltpu.sync_copy(shared_acc.at[pl.ds(sid * ZR, ZR)],
                    acc_out.at[pl.ds(cid * NACC + sid * ZR, ZR)])

    @pl.when(cid == 0)
    def _():
        pltpu.sync_copy(shared_deg.at[pl.ds(sid * ZR, ZR)],
                        deg_out.at[pl.ds(sid * ZR, ZR)])


_sc_aggregate = pl.kernel(
    _sc_body,
    out_type=(
        jax.ShapeDtypeStruct((NC * NACC, H), jnp.float32),
        jax.ShapeDtypeStruct((NACC, 8), jnp.float32),
    ),
    mesh=plsc.VectorSubcoreMesh(core_axis_name="c", subcore_axis_name="s"),
    scratch_types=(
        pltpu.VMEM_SHARED((NACC, H), jnp.float32),
        pltpu.VMEM_SHARED((NACC, 8), jnp.float32),
        pltpu.VMEM((STEPS, C), jnp.int32),
        pltpu.VMEM((STEPS, C), jnp.int32),
        pltpu.VMEM((NB, C, D), jnp.float32),
        pltpu.VMEM((C, 8), jnp.float32),
        pltpu.VMEM((C, 8), jnp.float32),
        pltpu.VMEM((C, H), jnp.float32),
        pltpu.SemaphoreType.DMA((NB,)),
        pltpu.SemaphoreType.DMA((NB,)),
        pltpu.SemaphoreType.DMA((NB,)),
    ),
    compiler_params=pltpu.CompilerParams(use_tc_tiling_on_sc=False),
)


def _tc_body(acc_ref, deg_ref, x_ref, wt_ref, b_ref, g_ref, be_ref, o_ref):
    summed = jnp.concatenate([acc_ref[:N, :], acc_ref[NACC:NACC + N, :]],
                             axis=1)
    deg = jnp.maximum(deg_ref[:N, 0:1], 1.0)
    h = summed / deg
    h = jnp.dot(h, wt_ref[...], preferred_element_type=jnp.float32)
    h = h + b_ref[...]
    mean = jnp.mean(h, axis=0, keepdims=True)
    var = jnp.mean((h - mean) * (h - mean), axis=0, keepdims=True)
    h = (h - mean) * jax.lax.rsqrt(var + 1e-5) * g_ref[...] + be_ref[...]
    o_ref[...] = x_ref[...] + jnp.maximum(h, 0.0)


_tc_tail = pl.pallas_call(
    _tc_body,
    out_shape=jax.ShapeDtypeStruct((N, D), jnp.float32),
)


def kernel(x, edge_index, W, b, gamma, beta):
    src = edge_index[0].astype(jnp.int32)
    dst = edge_index[1].astype(jnp.int32)
    pad = EPAD - E
    srcp = jnp.concatenate([src, jnp.full((pad,), N, jnp.int32)])
    dstp = jnp.concatenate([dst, jnp.full((pad,), N, jnp.int32)])
    src2 = jnp.concatenate([srcp, srcp + (N + 1)])
    src2d = src2.reshape(NC * NS * STEPS, C)
    dst2d = dstp.reshape(NS * STEPS, C)
    zrow = jnp.zeros((1, D), jnp.float32)
    xsplit = jnp.concatenate([x, zrow], axis=0)
    src2 = jnp.concatenate([jnp.minimum(srcp, N), jnp.minimum(srcp, N)])
    src2d = src2.reshape(NC * NS * STEPS, C)
    zacc = jnp.zeros((C, H), jnp.float32)
    zdeg = jnp.zeros((C, 8), jnp.float32)
    ones_h = jnp.ones((C, 8), jnp.float32)

    summed, deg = _sc_aggregate(xsplit, src2d, dst2d, zacc, zdeg, ones_h)

    return _tc_tail(summed, deg, x, W.T,
                    b.reshape(1, D), gamma.reshape(1, D), beta.reshape(1, D))


# speedup vs baseline: 2.0116x; 2.0116x over previous
import jax
import jax.numpy as jnp
from jax import lax
from jax.experimental import pallas as pl
from jax.experimental.pallas import tpu as pltpu
from jax.experimental.pallas import tpu_sc as plsc

N = 10000
E = 320000
D = 128
H = 64
NC = 2
NS = 16
C = 128
NB = 4
NI = 2 * NB
EPT = -(-E // (NS * C * NB)) * C * NB
EPAD = EPT * NS
STEPS = EPT // C
KMAX = (STEPS + NB) // NB
NACC = ((N + 1 + 127) // 128) * 128
ZR = NACC // NS


def _sc_body(xsplit, src2d, dst2d, zacc, zdeg, ones_h,
             acc_out, deg_out,
             shared_x, shared_acc, shared_deg,
             src_i, dst_i, rows, ones_v, zdeg_v, zacc_v,
             gsem, isem):
    cid = lax.axis_index("c")
    sid = lax.axis_index("s")

    pltpu.sync_copy(xsplit.at[pl.ds(cid * NACC + sid * ZR, ZR)],
                    shared_x.at[pl.ds(sid * ZR, ZR)])
    pltpu.sync_copy(zacc, zacc_v)
    pltpu.sync_copy(zdeg, zdeg_v)
    zbase = sid * ZR
    for i, sz in ((0, 128), (128, 128), (256, 128), (384, 128), (512, 120)):
        pltpu.sync_copy(zacc_v.at[pl.ds(0, sz)],
                        shared_acc.at[pl.ds(zbase + i, sz)])
        pltpu.sync_copy(zdeg_v.at[pl.ds(0, sz)],
                        shared_deg.at[pl.ds(zbase + i, sz)])
    pltpu.sync_copy(ones_h, ones_v)

    row0 = sid * STEPS
    for b in range(NI):
        pltpu.async_copy(src2d.at[pl.ds(row0 + b, 1)], src_i.at[pl.ds(b, 1)],
                         isem.at[b])
        pltpu.async_copy(dst2d.at[pl.ds(row0 + b, 1)], dst_i.at[pl.ds(b, 1)],
                         isem.at[b])

    plsc.subcore_barrier()

    def group(k, carry):
        for b in range(NB):
            j = k * NB + b

            @pl.when(j >= NB)
            def _():
                jp = j - NB
                ip = jp % NI
                pltpu.make_async_copy(
                    shared_x.at[src_i.at[ip]], rows.at[b],
                    gsem.at[b]).wait()
                pltpu.sync_copy(rows.at[b], shared_acc.at[dst_i.at[ip]],
                                add=True)
                pltpu.sync_copy(ones_v, shared_deg.at[dst_i.at[ip]],
                                add=True)

                @pl.when(j + NB < STEPS)
                def _():
                    jn = j + NB
                    inx = jn % NI
                    pltpu.async_copy(src2d.at[pl.ds(row0 + jn, 1)],
                                     src_i.at[pl.ds(inx, 1)], isem.at[inx])
                    pltpu.async_copy(dst2d.at[pl.ds(row0 + jn, 1)],
                                     dst_i.at[pl.ds(inx, 1)], isem.at[inx])

            @pl.when(j < STEPS)
            def _():
                inx = j % NI
                pltpu.make_async_copy(src2d.at[pl.ds(row0 + j, 1)],
                                      src_i.at[pl.ds(inx, 1)],
                                      isem.at[inx]).wait()
                pltpu.make_async_copy(dst2d.at[pl.ds(row0 + j, 1)],
                                      dst_i.at[pl.ds(inx, 1)],
                                      isem.at[inx]).wait()
                pltpu.async_copy(shared_x.at[src_i.at[inx]], rows.at[b],
                                 gsem.at[b])
        return carry

    lax.fori_loop(0, KMAX, group, 0)
    plsc.subcore_barrier()

    pltpu.sync_copy(shared_acc.at[pl.ds(sid * ZR, ZR)],
                    acc_out.at[pl.ds(cid * NACC + sid * ZR, ZR)])

    @pl.when(cid == 0)
    def _():
        pltpu.sync_copy(shared_deg.at[pl.ds(sid * ZR, ZR)],
                        deg_out.at[pl.ds(sid * ZR, ZR)])


_sc_aggregate = pl.kernel(
    _sc_body,
    out_type=(
        jax.ShapeDtypeStruct((NC * NACC, H), jnp.float32),
        jax.ShapeDtypeStruct((NACC, 8), jnp.float32),
    ),
    mesh=plsc.VectorSubcoreMesh(core_axis_name="c", subcore_axis_name="s"),
    scratch_types=(
        pltpu.VMEM_SHARED((NACC, H), jnp.float32),
        pltpu.VMEM_SHARED((NACC, H), jnp.float32),
        pltpu.VMEM_SHARED((NACC, 8), jnp.float32),
        pltpu.VMEM((NI, C), jnp.int32),
        pltpu.VMEM((NI, C), jnp.int32),
        pltpu.VMEM((NB, C, H), jnp.float32),
        pltpu.VMEM((C, 8), jnp.float32),
        pltpu.VMEM((C, 8), jnp.float32),
        pltpu.VMEM((C, H), jnp.float32),
        pltpu.SemaphoreType.DMA((NB,)),
        pltpu.SemaphoreType.DMA((NI,)),
    ),
    compiler_params=pltpu.CompilerParams(use_tc_tiling_on_sc=False),
)


def _tc_body(acc_ref, deg_ref, x_ref, wt_ref, b_ref, g_ref, be_ref, o_ref):
    summed = jnp.concatenate([acc_ref[:N, :], acc_ref[NACC:NACC + N, :]],
                             axis=1)
    deg = jnp.maximum(deg_ref[:N, 0:1], 1.0)
    h = summed / deg
    h = jnp.dot(h, wt_ref[...], preferred_element_type=jnp.float32)
    h = h + b_ref[...]
    mean = jnp.mean(h, axis=0, keepdims=True)
    var = jnp.mean((h - mean) * (h - mean), axis=0, keepdims=True)
    h = (h - mean) * jax.lax.rsqrt(var + 1e-5) * g_ref[...] + be_ref[...]
    o_ref[...] = x_ref[...] + jnp.maximum(h, 0.0)


_tc_tail = pl.pallas_call(
    _tc_body,
    out_shape=jax.ShapeDtypeStruct((N, D), jnp.float32),
)


def kernel(x, edge_index, W, b, gamma, beta):
    src = edge_index[0].astype(jnp.int32)
    dst = edge_index[1].astype(jnp.int32)
    pad = EPAD - E
    srcp = jnp.concatenate([src, jnp.full((pad,), N, jnp.int32)])
    dstp = jnp.concatenate([dst, jnp.full((pad,), N, jnp.int32)])
    src2d = srcp.reshape(NS * STEPS, C)
    dst2d = dstp.reshape(NS * STEPS, C)
    zpad = jnp.zeros((NACC - N, H), jnp.float32)
    xsplit = jnp.concatenate([x[:, :H], zpad, x[:, H:], zpad], axis=0)
    zacc = jnp.zeros((C, H), jnp.float32)
    zdeg = jnp.zeros((C, 8), jnp.float32)
    ones_h = jnp.ones((C, 8), jnp.float32)

    summed, deg = _sc_aggregate(xsplit, src2d, dst2d, zacc, zdeg, ones_h)

    return _tc_tail(summed, deg, x, W.T,
                    b.reshape(1, D), gamma.reshape(1, D), beta.reshape(1, D))


# speedup vs baseline: 2.3940x; 1.1901x over previous
import jax
import jax.numpy as jnp
from jax import lax
from jax.experimental import pallas as pl
from jax.experimental.pallas import tpu as pltpu
from jax.experimental.pallas import tpu_sc as plsc

N = 10000
E = 320000
D = 128
H = 64
NC = 2
NS = 16
C = 128
NB = 4
NI = 2 * NB
TCH = E // C
BASE = TCH // NS
REM = TCH - BASE * NS
NACC = ((N + 127) // 128) * 128
ZR = NACC // NS
XR = N // NS


def _sc_body(x, ei2, zacc, zdeg, ones_h,
             acc_out, deg_out,
             shared_x, shared_acc, shared_deg,
             src_i, dst_i, rows, ones_v, zdeg_v, zacc_v,
             gsem, isem):
    cid = lax.axis_index("c")
    sid = lax.axis_index("s")
    steps = BASE + jnp.where(sid < REM, 1, 0)

    pltpu.sync_copy(x.at[pl.ds(sid * XR, XR), pl.ds(cid * H, H)],
                    shared_x.at[pl.ds(sid * XR, XR)])
    pltpu.sync_copy(zacc, zacc_v)
    pltpu.sync_copy(zdeg, zdeg_v)
    zbase = sid * ZR
    for i, sz in ((0, 128), (128, 128), (256, 128), (384, 128), (512, 120)):
        pltpu.sync_copy(zacc_v.at[pl.ds(0, sz)],
                        shared_acc.at[pl.ds(zbase + i, sz)])
        pltpu.sync_copy(zdeg_v.at[pl.ds(0, sz)],
                        shared_deg.at[pl.ds(zbase + i, sz)])
    pltpu.sync_copy(ones_h, ones_v)

    for b in range(NI):
        pltpu.async_copy(ei2.at[0, pl.ds(sid + NS * b, 1)],
                         src_i.at[pl.ds(b, 1)], isem.at[b])
        pltpu.async_copy(ei2.at[1, pl.ds(sid + NS * b, 1)],
                         dst_i.at[pl.ds(b, 1)], isem.at[b])

    plsc.subcore_barrier()

    def group(k, carry):
        for b in range(NB):
            j = k * NB + b

            @pl.when(jnp.logical_and(j >= NB, j - NB < steps))
            def _():
                jp = j - NB
                ip = jp % NI
                pltpu.make_async_copy(
                    shared_x.at[src_i.at[ip]], rows.at[b],
                    gsem.at[b]).wait()
                pltpu.sync_copy(rows.at[b], shared_acc.at[dst_i.at[ip]],
                                add=True)
                pltpu.sync_copy(ones_v, shared_deg.at[dst_i.at[ip]],
                                add=True)

                @pl.when(j + NB < steps)
                def _():
                    jn = j + NB
                    inx = jn % NI
                    pltpu.async_copy(ei2.at[0, pl.ds(sid + NS * jn, 1)],
                                     src_i.at[pl.ds(inx, 1)], isem.at[inx])
                    pltpu.async_copy(ei2.at[1, pl.ds(sid + NS * jn, 1)],
                                     dst_i.at[pl.ds(inx, 1)], isem.at[inx])

            @pl.when(j < steps)
            def _():
                inx = j % NI
                pltpu.make_async_copy(ei2.at[0, pl.ds(sid + NS * j, 1)],
                                      src_i.at[pl.ds(inx, 1)],
                                      isem.at[inx]).wait()
                pltpu.make_async_copy(ei2.at[1, pl.ds(sid + NS * j, 1)],
                                      dst_i.at[pl.ds(inx, 1)],
                                      isem.at[inx]).wait()
                pltpu.async_copy(shared_x.at[src_i.at[inx]], rows.at[b],
                                 gsem.at[b])
        return carry

    kmax = (steps + 2 * NB - 1) // NB
    lax.fori_loop(0, kmax, group, 0)
    plsc.subcore_barrier()

    pltpu.sync_copy(shared_acc.at[pl.ds(sid * ZR, ZR)],
                    acc_out.at[pl.ds(cid * NACC + sid * ZR, ZR)])

    @pl.when(cid == 0)
    def _():
        pltpu.sync_copy(shared_deg.at[pl.ds(sid * ZR, ZR)],
                        deg_out.at[pl.ds(sid * ZR, ZR)])


_sc_aggregate = pl.kernel(
    _sc_body,
    out_type=(
        jax.ShapeDtypeStruct((NC * NACC, H), jnp.float32),
        jax.ShapeDtypeStruct((NACC, 8), jnp.float32),
    ),
    mesh=plsc.VectorSubcoreMesh(core_axis_name="c", subcore_axis_name="s"),
    scratch_types=(
        pltpu.VMEM_SHARED((N, H), jnp.float32),
        pltpu.VMEM_SHARED((NACC, H), jnp.float32),
        pltpu.VMEM_SHARED((NACC, 8), jnp.float32),
        pltpu.VMEM((NI, C), jnp.int32),
        pltpu.VMEM((NI, C), jnp.int32),
        pltpu.VMEM((NB, C, H), jnp.float32),
        pltpu.VMEM((C, 8), jnp.float32),
        pltpu.VMEM((C, 8), jnp.float32),
        pltpu.VMEM((C, H), jnp.float32),
        pltpu.SemaphoreType.DMA((NB,)),
        pltpu.SemaphoreType.DMA((NI,)),
    ),
    compiler_params=pltpu.CompilerParams(use_tc_tiling_on_sc=False),
)


def _tc_body(acc_ref, deg_ref, x_ref, wt_ref, b_ref, g_ref, be_ref, o_ref):
    summed = jnp.concatenate([acc_ref[:N, :], acc_ref[NACC:NACC + N, :]],
                             axis=1)
    deg = jnp.maximum(deg_ref[:N, 0:1], 1.0)
    h = summed / deg
    h = jnp.dot(h, wt_ref[...], preferred_element_type=jnp.float32)
    h = h + b_ref[...]
    mean = jnp.mean(h, axis=0, keepdims=True)
    var = jnp.mean((h - mean) * (h - mean), axis=0, keepdims=True)
    h = (h - mean) * jax.lax.rsqrt(var + 1e-5) * g_ref[...] + be_ref[...]
    o_ref[...] = x_ref[...] + jnp.maximum(h, 0.0)


_tc_tail = pl.pallas_call(
    _tc_body,
    out_shape=jax.ShapeDtypeStruct((N, D), jnp.float32),
)


def kernel(x, edge_index, W, b, gamma, beta):
    ei2 = edge_index.astype(jnp.int32).reshape(2, TCH, C)
    zacc = jnp.zeros((C, H), jnp.float32)
    zdeg = jnp.zeros((C, 8), jnp.float32)
    ones_h = jnp.ones((C, 8), jnp.float32)

    summed, deg = _sc_aggregate(x, ei2, zacc, zdeg, ones_h)

    return _tc_tail(summed, deg, x, W.T,
                    b.reshape(1, D), gamma.reshape(1, D), beta.reshape(1, D))


# speedup vs baseline: 2.4673x; 1.0306x over previous
import jax
import jax.numpy as jnp
from jax import lax
from jax.experimental import pallas as pl
from jax.experimental.pallas import tpu as pltpu
from jax.experimental.pallas import tpu_sc as plsc

N = 10000
E = 320000
D = 128
H = 64
NC = 2
NS = 16
C = 128
NB = 4
NI = 2 * NB
TCH = E // C
BASE = TCH // NS
REM = TCH - BASE * NS
NACC = ((N + 127) // 128) * 128
ZR = NACC // NS
XR = N // NS


def _sc_body(x, ei2, zacc, zdeg, ones_h,
             acc_out, deg_out,
             shared_x, shared_acc, shared_deg,
             src_i, dst_i, rows, ones_v, zdeg_v, zacc_v,
             gsem, isem):
    cid = lax.axis_index("c")
    sid = lax.axis_index("s")
    steps = BASE + jnp.where(sid < REM, 1, 0)

    for b in range(NI):
        pltpu.async_copy(ei2.at[0, pl.ds(sid + NS * b, 1)],
                         src_i.at[pl.ds(b, 1)], isem.at[b])
        pltpu.async_copy(ei2.at[1, pl.ds(sid + NS * b, 1)],
                         dst_i.at[pl.ds(b, 1)], isem.at[b])

    pltpu.sync_copy(x.at[pl.ds(sid * XR, XR), pl.ds(cid * H, H)],
                    shared_x.at[pl.ds(sid * XR, XR)])
    pltpu.sync_copy(zacc, zacc_v)
    pltpu.sync_copy(zdeg, zdeg_v)
    zbase = sid * ZR
    for i, sz in ((0, 128), (128, 128), (256, 128), (384, 128), (512, 120)):
        pltpu.sync_copy(zacc_v.at[pl.ds(0, sz)],
                        shared_acc.at[pl.ds(zbase + i, sz)])
        pltpu.sync_copy(zdeg_v.at[pl.ds(0, sz)],
                        shared_deg.at[pl.ds(zbase + i, sz)])
    pltpu.sync_copy(ones_h, ones_v)

    plsc.subcore_barrier()

    def group(k, carry):
        for b in range(NB):
            j = k * NB + b

            @pl.when(jnp.logical_and(j >= NB, j - NB < steps))
            def _():
                jp = j - NB
                ip = jp % NI
                pltpu.make_async_copy(
                    shared_x.at[src_i.at[ip]], rows.at[b],
                    gsem.at[b]).wait()
                pltpu.sync_copy(rows.at[b], shared_acc.at[dst_i.at[ip]],
                                add=True)
                if b % 2 == 0:
                    @pl.when(cid == 0)
                    def _():
                        pltpu.sync_copy(ones_v, shared_deg.at[dst_i.at[ip]],
                                        add=True)
                else:
                    @pl.when(cid == 1)
                    def _():
                        pltpu.sync_copy(ones_v, shared_deg.at[dst_i.at[ip]],
                                        add=True)

                @pl.when(j + NB < steps)
                def _():
                    jn = j + NB
                    inx = jn % NI
                    pltpu.async_copy(ei2.at[0, pl.ds(sid + NS * jn, 1)],
                                     src_i.at[pl.ds(inx, 1)], isem.at[inx])
                    pltpu.async_copy(ei2.at[1, pl.ds(sid + NS * jn, 1)],
                                     dst_i.at[pl.ds(inx, 1)], isem.at[inx])

            @pl.when(j < steps)
            def _():
                inx = j % NI
                pltpu.make_async_copy(ei2.at[0, pl.ds(sid + NS * j, 1)],
                                      src_i.at[pl.ds(inx, 1)],
                                      isem.at[inx]).wait()
                pltpu.make_async_copy(ei2.at[1, pl.ds(sid + NS * j, 1)],
                                      dst_i.at[pl.ds(inx, 1)],
                                      isem.at[inx]).wait()
                pltpu.async_copy(shared_x.at[src_i.at[inx]], rows.at[b],
                                 gsem.at[b])
        return carry

    kmax = (steps + 2 * NB - 1) // NB
    lax.fori_loop(0, kmax, group, 0)
    plsc.subcore_barrier()

    pltpu.sync_copy(shared_acc.at[pl.ds(sid * ZR, ZR)],
                    acc_out.at[pl.ds(cid * NACC + sid * ZR, ZR)])
    pltpu.sync_copy(shared_deg.at[pl.ds(sid * ZR, ZR)],
                    deg_out.at[pl.ds(cid * NACC + sid * ZR, ZR)])


_sc_aggregate = pl.kernel(
    _sc_body,
    out_type=(
        jax.ShapeDtypeStruct((NC * NACC, H), jnp.float32),
        jax.ShapeDtypeStruct((NC * NACC, 8), jnp.float32),
    ),
    mesh=plsc.VectorSubcoreMesh(core_axis_name="c", subcore_axis_name="s"),
    scratch_types=(
        pltpu.VMEM_SHARED((N, H), jnp.float32),
        pltpu.VMEM_SHARED((NACC, H), jnp.float32),
        pltpu.VMEM_SHARED((NACC, 8), jnp.float32),
        pltpu.VMEM((NI, C), jnp.int32),
        pltpu.VMEM((NI, C), jnp.int32),
        pltpu.VMEM((NB, C, H), jnp.float32),
        pltpu.VMEM((C, 8), jnp.float32),
        pltpu.VMEM((C, 8), jnp.float32),
        pltpu.VMEM((C, H), jnp.float32),
        pltpu.SemaphoreType.DMA((NB,)),
        pltpu.SemaphoreType.DMA((NI,)),
    ),
    compiler_params=pltpu.CompilerParams(use_tc_tiling_on_sc=False),
)


def _tc_body(acc_ref, deg_ref, x_ref, wt_ref, b_ref, g_ref, be_ref, o_ref):
    summed = jnp.concatenate([acc_ref[:N, :], acc_ref[NACC:NACC + N, :]],
                             axis=1)
    deg = jnp.maximum(deg_ref[:N, 0:1] + deg_ref[NACC:NACC + N, 0:1], 1.0)
    h = summed / deg
    h = jnp.dot(h, wt_ref[...], preferred_element_type=jnp.float32)
    h = h + b_ref[...]
    mean = jnp.mean(h, axis=0, keepdims=True)
    var = jnp.mean((h - mean) * (h - mean), axis=0, keepdims=True)
    h = (h - mean) * jax.lax.rsqrt(var + 1e-5) * g_ref[...] + be_ref[...]
    o_ref[...] = x_ref[...] + jnp.maximum(h, 0.0)


_tc_tail = pl.pallas_call(
    _tc_body,
    out_shape=jax.ShapeDtypeStruct((N, D), jnp.float32),
)


def kernel(x, edge_index, W, b, gamma, beta):
    ei2 = edge_index.astype(jnp.int32).reshape(2, TCH, C)
    zacc = jnp.zeros((C, H), jnp.float32)
    zdeg = jnp.zeros((C, 8), jnp.float32)
    ones_h = jnp.ones((C, 8), jnp.float32)

    summed, deg = _sc_aggregate(x, ei2, zacc, zdeg, ones_h)

    return _tc_tail(summed, deg, x, W.T,
                    b.reshape(1, D), gamma.reshape(1, D), beta.reshape(1, D))
